# Initial kernel scaffold; baseline (speedup 1.0000x reference)
import jax
import jax.numpy as jnp
from jax.experimental import pallas as pl


def kernel(inputs, targets, predict_features, global_p1_features, global_p2_features):
    raise NotImplementedError("write your pallas kernel here")


# speedup vs baseline: 2.7348x; 2.7348x over previous
import functools

import jax
import jax.numpy as jnp
from jax import lax
from jax.experimental import pallas as pl
from jax.experimental.pallas import tpu as pltpu

TEMP = 0.05
_KB = 256


def _main_body(x_ref, t_ref, f0_ref, f1_ref, f2_ref, out_ref,
               xs_ref, s_refs, g_refs, *, K, nkb):
    B = x_ref.shape[1]
    k = pl.program_id(0)

    @pl.when(k == 0)
    def _init():
        for i in range(3):
            x = x_ref[i]
            n = jnp.sqrt(jnp.sum(x * x, axis=1, keepdims=True))
            xs_ref[i] = x / jnp.maximum(n, 1e-12) * (1.0 / TEMP)
        for r in s_refs + g_refs:
            r[...] = jnp.zeros_like(r)

    col0 = k * _KB
    iota = lax.broadcasted_iota(jnp.int32, (B, _KB), 1) + col0
    valid = iota < K
    tgt = t_ref[...]
    for i, f_ref in enumerate((f0_ref, f1_ref, f2_ref)):
        xs = xs_ref[i]
        logits = lax.dot_general(xs, f_ref[...], (((1,), (1,)), ((), ())),
                                 preferred_element_type=jnp.float32)
        logits = jnp.where(valid, logits, -1e30)
        s_refs[i][...] += jnp.sum(jnp.exp(logits), axis=1, keepdims=True)
        g_refs[i][...] += jnp.sum(jnp.where(iota == tgt, logits, 0.0),
                                  axis=1, keepdims=True)

    @pl.when(k == nkb - 1)
    def _fini():
        ce = 0.0
        for i in range(3):
            ce += jnp.sum(jnp.log(s_refs[i][...]) - g_refs[i][...]) / B
        x1 = xs_ref[1] * TEMP
        x2 = xs_ref[2] * TEMP
        m1 = jnp.max(x1, axis=1, keepdims=True)
        m2 = jnp.max(x2, axis=1, keepdims=True)
        e1 = jnp.exp(x1 - m1)
        e2 = jnp.exp(x2 - m2)
        z1 = jnp.sum(e1, axis=1, keepdims=True)
        z2 = jnp.sum(e2, axis=1, keepdims=True)
        p1 = e1 / z1
        p2 = e2 / z2
        lp1 = (x1 - m1) - jnp.log(z1)
        lp2 = (x2 - m2) - jnp.log(z2)
        lm = jnp.log((p1 + p2) * 0.5)
        kl1 = jnp.sum(p1 * (lp1 - lm))
        kl2 = jnp.sum(p2 * (lp2 - lm))
        out_ref[0, 0] = ce + 0.5 * (kl1 + kl2)


def _run(inputs, targets, f0, f1, f2):
    _, B, D = inputs.shape
    K = f0.shape[0]
    nkb = pl.cdiv(K, _KB)
    t2d = targets.astype(jnp.int32).reshape(B, 1)

    def body(x_ref, t_ref, f0_ref, f1_ref, f2_ref, out_ref, xs_ref,
             s0, s1, s2, g0, g1, g2):
        _main_body(x_ref, t_ref, f0_ref, f1_ref, f2_ref, out_ref,
                   xs_ref, [s0, s1, s2], [g0, g1, g2], K=K, nkb=nkb)

    out = pl.pallas_call(
        body,
        grid=(nkb,),
        in_specs=[
            pl.BlockSpec((3, B, D), lambda k: (0, 0, 0)),
            pl.BlockSpec((B, 1), lambda k: (0, 0)),
            pl.BlockSpec((_KB, D), lambda k: (k, 0)),
            pl.BlockSpec((_KB, D), lambda k: (k, 0)),
            pl.BlockSpec((_KB, D), lambda k: (k, 0)),
        ],
        out_specs=pl.BlockSpec(memory_space=pltpu.SMEM),
        out_shape=jax.ShapeDtypeStruct((1, 1), jnp.float32),
        scratch_shapes=[pltpu.VMEM((3, B, D), jnp.float32)]
        + [pltpu.VMEM((B, 1), jnp.float32) for _ in range(6)],
        compiler_params=pltpu.CompilerParams(
            dimension_semantics=("arbitrary",)),
    )(inputs, t2d, f0, f1, f2)
    return out[0, 0]


def kernel(inputs, targets, predict_features, global_p1_features,
           global_p2_features):
    return _run(inputs, targets, predict_features, global_p1_features,
                global_p2_features)
